# baseline (speedup 1.0000x reference)
import functools

import jax
import jax.numpy as jnp
from jax import lax
from jax.experimental import pallas as pl
from jax.experimental.pallas import tpu as pltpu
from jax.experimental.pallas import tpu_sc as plsc

BLK = 64
D = 1024
L = 16
UNITS_PER_WORKER = 16


def _sc_body(states, maskr, diff, mean, cmask, xbuf, mbuf, meanbuf, cmbuf):
    wid = lax.axis_index("s") * 2 + lax.axis_index("c")
    lanes = lax.iota(jnp.int32, L)

    def unit(i, cmvec):
        u = wid * UNITS_PER_WORKER + i
        pltpu.sync_copy(states.at[u], xbuf)
        pltpu.sync_copy(maskr.at[u], mbuf)
        mchunks = [mbuf[pl.ds(k * L, L)] for k in range(BLK // L)]
        ms = [mchunks[r // L][r % L] for r in range(BLK)]
        cnt = functools.reduce(lambda a, b: a + b, ms)
        denom = jnp.full((L,), cnt, jnp.float32) + 1e-4
        rcp = 1.0 / denom

        def col(c, carry):
            o = pl.multiple_of(c * L, L)
            acc = jnp.zeros((L,), jnp.float32)
            for r in range(BLK):
                xm = xbuf[r, pl.ds(o, L)] * ms[r]
                xbuf[r, pl.ds(o, L)] = xm
                acc = acc + xm
            mc = acc * rcp
            meanbuf[pl.ds(o, L)] = mc
            for r in range(BLK):
                xbuf[r, pl.ds(o, L)] = mc - xbuf[r, pl.ds(o, L)]
            return carry

        lax.fori_loop(0, D // L, col, 0)
        pltpu.sync_copy(xbuf, diff.at[u])
        pltpu.sync_copy(meanbuf, mean.at[u])
        cmval = (cnt > 0.0).astype(jnp.float32)
        return jnp.where(lanes == i, cmval, cmvec)

    cmvec = lax.fori_loop(0, UNITS_PER_WORKER, unit, jnp.zeros((L,), jnp.float32))
    cmbuf[...] = cmvec
    pltpu.sync_copy(cmbuf, cmask.at[pl.ds(wid * UNITS_PER_WORKER, UNITS_PER_WORKER)])


@jax.jit
def _run(states, mask):
    b, f, d = states.shape
    nb = f // BLK
    n = b * nb
    sr = states.reshape(n, BLK, d)
    mr = mask.reshape(n, BLK)
    mesh = plsc.VectorSubcoreMesh(core_axis_name="c", subcore_axis_name="s")
    diff, mean, cmask = pl.kernel(
        _sc_body,
        mesh=mesh,
        out_type=[
            jax.ShapeDtypeStruct((n, BLK, d), states.dtype),
            jax.ShapeDtypeStruct((n, d), states.dtype),
            jax.ShapeDtypeStruct((n,), states.dtype),
        ],
        scratch_types=[
            pltpu.VMEM((BLK, d), jnp.float32),
            pltpu.VMEM((BLK,), jnp.float32),
            pltpu.VMEM((d,), jnp.float32),
            pltpu.VMEM((UNITS_PER_WORKER,), jnp.float32),
        ],
    )(sr, mr)
    return diff, mean, cmask


def kernel(fine_token_states, fine_token_mask):
    b, f, d = fine_token_states.shape
    nb = f // BLK
    diff, mean, cmask = _run(fine_token_states, fine_token_mask)
    indice = jnp.broadcast_to(jnp.arange(nb, dtype=jnp.int32)[None, :], (b, nb))
    return (mean.reshape(b, nb, d), cmask.reshape(b, nb),
            diff.reshape(b, nb, BLK, d), indice)


# speedup vs baseline: 1.3316x; 1.3316x over previous
import functools

import jax
import jax.numpy as jnp
from jax import lax
from jax.experimental import pallas as pl
from jax.experimental.pallas import tpu as pltpu
from jax.experimental.pallas import tpu_sc as plsc

BLK = 64
D = 1024
W = 512
L = 16
UPW = 16


def _sc_body(states, maskr, diff, mean, cmask,
             xb0, xb1, mall, meanbuf, cmbuf, sem0, sem1):
    wid = lax.axis_index("s") * 2 + lax.axis_index("c")
    lanes = lax.iota(jnp.int32, L)
    base = wid * UPW

    pltpu.sync_copy(maskr.at[pl.ds(base, UPW)], mall)
    pltpu.async_copy(states.at[base, :, pl.ds(0, W)], xb0, sem0)

    def compute(buf, ms, rcp, mo):
        def col(c, carry):
            o = pl.multiple_of(c * L, L)
            acc = [jnp.zeros((L,), jnp.float32) for _ in range(4)]
            for r in range(BLK):
                xm = buf[r, pl.ds(o, L)] * ms[r]
                buf[r, pl.ds(o, L)] = xm
                acc[r % 4] = acc[r % 4] + xm
            mc = ((acc[0] + acc[1]) + (acc[2] + acc[3])) * rcp
            meanbuf[pl.ds(mo + o, L)] = mc
            for r in range(BLK):
                buf[r, pl.ds(o, L)] = mc - buf[r, pl.ds(o, L)]
            return carry

        lax.fori_loop(0, W // L, col, 0, unroll=False)

    def unit(i, cmvec):
        u = base + i
        mchunks = [mall[i, pl.ds(k * L, L)] for k in range(BLK // L)]
        ms = [mchunks[r // L][r % L] for r in range(BLK)]
        cnt = functools.reduce(lambda a, b: a + b, ms)
        denom = jnp.full((L,), cnt, jnp.float32) + 1e-4
        rcp = 1.0 / denom

        pltpu.make_async_copy(states.at[u, :, pl.ds(0, W)], xb0, sem0).wait()
        pltpu.async_copy(states.at[u, :, pl.ds(W, W)], xb1, sem1)
        compute(xb0, ms, rcp, 0)
        pltpu.sync_copy(xb0, diff.at[u, :, pl.ds(0, W)])

        pltpu.make_async_copy(states.at[u, :, pl.ds(W, W)], xb1, sem1).wait()

        @pl.when(i + 1 < UPW)
        def _():
            pltpu.async_copy(states.at[u + 1, :, pl.ds(0, W)], xb0, sem0)

        compute(xb1, ms, rcp, W)
        pltpu.sync_copy(xb1, diff.at[u, :, pl.ds(W, W)])
        pltpu.sync_copy(meanbuf, mean.at[u])

        cmval = (cnt > 0.0).astype(jnp.float32)
        return jnp.where(lanes == i, cmval, cmvec)

    cmvec = lax.fori_loop(0, UPW, unit, jnp.zeros((L,), jnp.float32))
    cmbuf[...] = cmvec
    pltpu.sync_copy(cmbuf, cmask.at[pl.ds(base, UPW)])


@jax.jit
def _run(states, mask):
    b, f, d = states.shape
    nb = f // BLK
    n = b * nb
    sr = states.reshape(n, BLK, d)
    mr = mask.reshape(n, BLK)
    mesh = plsc.VectorSubcoreMesh(core_axis_name="c", subcore_axis_name="s")
    diff, mean, cmask = pl.kernel(
        _sc_body,
        mesh=mesh,
        out_type=[
            jax.ShapeDtypeStruct((n, BLK, d), states.dtype),
            jax.ShapeDtypeStruct((n, d), states.dtype),
            jax.ShapeDtypeStruct((n,), states.dtype),
        ],
        scratch_types=[
            pltpu.VMEM((BLK, W), jnp.float32),
            pltpu.VMEM((BLK, W), jnp.float32),
            pltpu.VMEM((UPW, BLK), jnp.float32),
            pltpu.VMEM((d,), jnp.float32),
            pltpu.VMEM((L,), jnp.float32),
            pltpu.SemaphoreType.DMA,
            pltpu.SemaphoreType.DMA,
        ],
    )(sr, mr)
    return diff, mean, cmask


def kernel(fine_token_states, fine_token_mask):
    b, f, d = fine_token_states.shape
    nb = f // BLK
    diff, mean, cmask = _run(fine_token_states, fine_token_mask)
    indice = jnp.broadcast_to(jnp.arange(nb, dtype=jnp.int32)[None, :], (b, nb))
    return (mean.reshape(b, nb, d), cmask.reshape(b, nb),
            diff.reshape(b, nb, BLK, d), indice)
